# baseline (speedup 1.0000x reference)
import jax
import jax.numpy as jnp
from jax import lax
from jax.experimental import pallas as pl

R = 32
A = 8
N = 4096
EPS = 1e-12


def _km_body(xt_ref, m_ref, sa_ref, sb_ref, c1_ref, c2_ref,
             c1c_ref, c2c_ref, out_ref):
    xt = xt_ref[...]
    m = m_ref[...]
    sa = sa_ref[...]
    sb = sb_ref[...]
    c1r = c1_ref[...]
    c2r = c2_ref[...]
    c1col = c1c_ref[...]
    c2col = c2c_ref[...]

    sbig = jnp.maximum(sa, sb)
    ssml = jnp.minimum(sa, sb)
    wu = -0.5 / (sbig * sbig)
    wl = -0.5 / (ssml * ssml)

    x2t = xt * xt
    f32 = jnp.float32
    ku = jnp.sum(wu * m * m, axis=1, keepdims=True)
    kl = jnp.sum(wl * m * m, axis=1, keepdims=True)
    au = (jnp.dot(wu, x2t, preferred_element_type=f32)
          + jnp.dot(-2.0 * m * wu, xt, preferred_element_type=f32) + ku)
    al = (jnp.dot(wl, x2t, preferred_element_type=f32)
          + jnp.dot(-2.0 * m * wl, xt, preferred_element_type=f32) + kl)
    uu = jnp.exp(au)
    ll = jnp.exp(al)
    dlt = uu - ll

    io = lax.broadcasted_iota(jnp.int32, (R, R), 0)
    ii = lax.broadcasted_iota(jnp.int32, (R, R), 1)

    def rank_row(cr, ccol):
        win = (ccol < cr) | ((ccol == cr) & (io < ii))
        return jnp.sum(win.astype(jnp.int32), axis=0, keepdims=True)

    rk1 = rank_row(c1r, c1col)
    rk2 = rank_row(c2r, c2col)
    m1 = (io >= rk1).astype(f32)
    m2 = (io >= rk2).astype(f32)

    s0l = jnp.sum(c1col * ll, axis=0, keepdims=True)
    t0l = jnp.sum(ll, axis=0, keepdims=True)
    s0r = jnp.sum(c2col * uu, axis=0, keepdims=True)
    t0r = jnp.sum(uu, axis=0, keepdims=True)

    s_cum = jnp.dot(m1 * c1r, dlt, preferred_element_type=f32)
    t_cum = jnp.dot(m1, dlt, preferred_element_type=f32)
    ratl = (s0l + s_cum) / (t0l + t_cum + EPS)
    lmin = jnp.minimum(jnp.min(ratl, axis=0, keepdims=True),
                       s0l / (t0l + EPS))

    s_cum2 = jnp.dot(m2 * c2r, dlt, preferred_element_type=f32)
    t_cum2 = jnp.dot(m2, dlt, preferred_element_type=f32)
    ratr = (s0r - s_cum2) / (t0r - t_cum2 + EPS)
    rmax = jnp.maximum(jnp.max(ratr, axis=0, keepdims=True),
                       s0r / (t0r + EPS))

    out_ref[...] = (lmin + rmax) * 0.5


_km_call = pl.pallas_call(
    _km_body,
    out_shape=jax.ShapeDtypeStruct((1, N), jnp.float32),
)


@jax.jit
def kernel(input_data, FRB_weights, c1, c2):
    xt = input_data.T
    m = FRB_weights[0:R * A].reshape(R, A)
    sa = FRB_weights[1:R * A + 1].reshape(R, A)
    sb = FRB_weights[2:R * A + 2].reshape(R, A)
    y = _km_call(xt, m, sa, sb, c1.reshape(1, R), c2.reshape(1, R),
                 c1.reshape(R, 1), c2.reshape(R, 1))
    return y.reshape(N)
